# baseline (device time: 124708 ns/iter reference)
import jax
import jax.numpy as jnp
from jax import lax
from jax.experimental import pallas as pl
from jax.experimental.pallas import tpu as pltpu

N_DEV = 8


def kernel(x, router_W, route_idx, expert_W, shared_W):
    n_tok, d = x.shape
    n_exp = router_W.shape[1]
    e_loc, _, h = expert_W.shape
    chunk = n_tok // N_DEV
    n_steps = N_DEV - 1

    def body(x_ref, rw_ref, idx_ref, ew_ref, sw_ref, out_ref,
             partial_ref, rs_buf, ag_buf,
             rs_send, rs_recv, ag_send, ag_recv):
        my = lax.axis_index("i")
        left = (my + N_DEV - 1) % N_DEV
        right = (my + 1) % N_DEV

        xv = x_ref[:, :]

        scores = jnp.dot(xv, rw_ref[:, :], preferred_element_type=jnp.float32)
        m = jnp.max(scores, axis=-1, keepdims=True)
        p = jnp.exp(scores - m)
        probs = p / jnp.sum(p, axis=-1, keepdims=True)
        ridx = idx_ref[:, 0:1]
        e_ids = lax.broadcasted_iota(jnp.int32, (n_tok, n_exp), 1)
        p_sel = jnp.sum(jnp.where(e_ids == ridx, probs, 0.0),
                        axis=1, keepdims=True)

        acc = jnp.zeros((n_tok, h), jnp.float32)
        for k in range(e_loc):
            e_g = my * e_loc + k
            coeff = jnp.where(ridx == e_g, p_sel, 0.0)
            y = jnp.dot(xv, ew_ref[k], preferred_element_type=jnp.float32)
            acc = acc + coeff * y
        partial_ref[:, :] = acc

        out_ref[:, :] = jnp.dot(xv, sw_ref[:, :],
                                preferred_element_type=jnp.float32)

        barrier_sem = pltpu.get_barrier_semaphore()
        pl.semaphore_signal(barrier_sem, inc=1, device_id=(left,),
                            device_id_type=pl.DeviceIdType.MESH)
        pl.semaphore_signal(barrier_sem, inc=1, device_id=(right,),
                            device_id_type=pl.DeviceIdType.MESH)
        pl.semaphore_wait(barrier_sem, 2)

        for s in range(n_steps):
            c_send = (my + N_DEV - s) % N_DEV
            rdma = pltpu.make_async_remote_copy(
                src_ref=partial_ref.at[pl.ds(c_send * chunk, chunk), :],
                dst_ref=rs_buf.at[s],
                send_sem=rs_send.at[s],
                recv_sem=rs_recv.at[s],
                device_id=(right,),
                device_id_type=pl.DeviceIdType.MESH,
            )
            rdma.start()
            rdma.wait()
            c_recv = (my + N_DEV - s - 1) % N_DEV
            rows = pl.ds(c_recv * chunk, chunk)
            partial_ref[rows, :] = partial_ref[rows, :] + rs_buf[s]

        c_own = (my + 1) % N_DEV
        own_rows = pl.ds(c_own * chunk, chunk)
        out_ref[own_rows, :] = out_ref[own_rows, :] + partial_ref[own_rows, :]

        for s in range(n_steps):
            if s == 0:
                src = partial_ref.at[pl.ds(c_own * chunk, chunk), :]
            else:
                src = ag_buf.at[s - 1]
            rdma = pltpu.make_async_remote_copy(
                src_ref=src,
                dst_ref=ag_buf.at[s],
                send_sem=ag_send.at[s],
                recv_sem=ag_recv.at[s],
                device_id=(right,),
                device_id_type=pl.DeviceIdType.MESH,
            )
            rdma.start()
            rdma.wait()
            c_recv = (my + N_DEV - s) % N_DEV
            rows = pl.ds(c_recv * chunk, chunk)
            out_ref[rows, :] = out_ref[rows, :] + ag_buf[s]

    return pl.pallas_call(
        body,
        out_shape=jax.ShapeDtypeStruct((n_tok, h), jnp.float32),
        in_specs=[pl.BlockSpec(memory_space=pltpu.VMEM)] * 5,
        out_specs=pl.BlockSpec(memory_space=pltpu.VMEM),
        scratch_shapes=[
            pltpu.VMEM((n_tok, h), jnp.float32),
            pltpu.VMEM((n_steps, chunk, h), jnp.float32),
            pltpu.VMEM((n_steps, chunk, h), jnp.float32),
            pltpu.SemaphoreType.DMA((n_steps,)),
            pltpu.SemaphoreType.DMA((n_steps,)),
            pltpu.SemaphoreType.DMA((n_steps,)),
            pltpu.SemaphoreType.DMA((n_steps,)),
        ],
        compiler_params=pltpu.CompilerParams(collective_id=0),
    )(x, router_W, route_idx, expert_W, shared_W)


# device time: 86195 ns/iter; 1.4468x vs baseline; 1.4468x over previous
import jax
import jax.numpy as jnp
from jax import lax
from jax.experimental import pallas as pl
from jax.experimental.pallas import tpu as pltpu

N_DEV = 8


def kernel(x, router_W, route_idx, expert_W, shared_W):
    n_tok, d = x.shape
    n_exp = router_W.shape[1]
    e_loc, _, h = expert_W.shape
    chunk = n_tok // N_DEV
    n_steps = N_DEV - 1
    h2 = h // 2

    def body(x_ref, rw_ref, idx_ref, ew_ref, sw_ref, out_ref,
             partial_ref, rsR, rsL, agR, agL,
             rsR_s, rsR_r, rsL_s, rsL_r, agR_s, agR_r, agL_s, agL_r):
        my = lax.axis_index("i")
        left = (my + N_DEV - 1) % N_DEV
        right = (my + 1) % N_DEV

        def rows(c):
            return pl.ds(c * chunk, chunk)

        def mk(src, dst, ssem, rsem, dev):
            return pltpu.make_async_remote_copy(
                src_ref=src, dst_ref=dst, send_sem=ssem, recv_sem=rsem,
                device_id=(dev,), device_id_type=pl.DeviceIdType.MESH)

        xv = x_ref[:, :]

        scores = jnp.dot(xv, rw_ref[:, :], preferred_element_type=jnp.float32)
        m = jnp.max(scores, axis=-1, keepdims=True)
        p = jnp.exp(scores - m)
        probs = p / jnp.sum(p, axis=-1, keepdims=True)
        ridx = idx_ref[:, 0:1]
        e_ids = lax.broadcasted_iota(jnp.int32, (n_tok, n_exp), 1)
        p_sel = jnp.sum(jnp.where(e_ids == ridx, probs, 0.0),
                        axis=1, keepdims=True)

        acc = jnp.zeros((n_tok, h), jnp.float32)
        for k in range(e_loc):
            e_g = my * e_loc + k
            coeff = jnp.where(ridx == e_g, p_sel, 0.0)
            y = jnp.dot(xv, ew_ref[k], preferred_element_type=jnp.float32)
            acc = acc + coeff * y
        partial_ref[:, :] = acc

        out_ref[:, :] = jnp.dot(xv, sw_ref[:, :],
                                preferred_element_type=jnp.float32)

        barrier_sem = pltpu.get_barrier_semaphore()
        pl.semaphore_signal(barrier_sem, inc=1, device_id=(left,),
                            device_id_type=pl.DeviceIdType.MESH)
        pl.semaphore_signal(barrier_sem, inc=1, device_id=(right,),
                            device_id_type=pl.DeviceIdType.MESH)
        pl.semaphore_wait(barrier_sem, 2)

        mk(partial_ref.at[rows(my), pl.ds(0, h2)], rsR.at[0],
           rsR_s.at[0], rsR_r.at[0], right).start()
        mk(partial_ref.at[rows(my), pl.ds(h2, h2)], rsL.at[0],
           rsL_s.at[0], rsL_r.at[0], left).start()

        for s in range(n_steps):
            mk(rsR.at[s], rsR.at[s], rsR_s.at[s], rsR_r.at[s],
               right).wait_recv()
            cR = (my + N_DEV - s - 1) % N_DEV
            rsR[s] = rsR[s] + partial_ref[rows(cR), 0:h2]
            if s < n_steps - 1:
                mk(rsR.at[s], rsR.at[s + 1], rsR_s.at[s + 1],
                   rsR_r.at[s + 1], right).start()

            mk(rsL.at[s], rsL.at[s], rsL_s.at[s], rsL_r.at[s],
               left).wait_recv()
            cL = (my + s + 1) % N_DEV
            rsL[s] = rsL[s] + partial_ref[rows(cL), h2:h]
            if s < n_steps - 1:
                mk(rsL.at[s], rsL.at[s + 1], rsL_s.at[s + 1],
                   rsL_r.at[s + 1], left).start()

        c_ownR = (my + 1) % N_DEV
        c_ownL = (my + N_DEV - 1) % N_DEV
        mk(rsR.at[n_steps - 1], agR.at[0], agR_s.at[0], agR_r.at[0],
           right).start()
        mk(rsL.at[n_steps - 1], agL.at[0], agL_s.at[0], agL_r.at[0],
           left).start()
        out_ref[rows(c_ownR), 0:h2] = (
            out_ref[rows(c_ownR), 0:h2] + rsR[n_steps - 1])
        out_ref[rows(c_ownL), h2:h] = (
            out_ref[rows(c_ownL), h2:h] + rsL[n_steps - 1])

        for s in range(n_steps):
            mk(agR.at[s], agR.at[s], agR_s.at[s], agR_r.at[s],
               right).wait_recv()
            if s < n_steps - 1:
                mk(agR.at[s], agR.at[s + 1], agR_s.at[s + 1],
                   agR_r.at[s + 1], right).start()
            cR = (my + N_DEV - s) % N_DEV
            out_ref[rows(cR), 0:h2] = out_ref[rows(cR), 0:h2] + agR[s]

            mk(agL.at[s], agL.at[s], agL_s.at[s], agL_r.at[s],
               left).wait_recv()
            if s < n_steps - 1:
                mk(agL.at[s], agL.at[s + 1], agL_s.at[s + 1],
                   agL_r.at[s + 1], left).start()
            cL = (my + s) % N_DEV
            out_ref[rows(cL), h2:h] = out_ref[rows(cL), h2:h] + agL[s]

        for s in range(n_steps):
            mk(rsR.at[s], rsR.at[s], rsR_s.at[s], rsR_r.at[s],
               right).wait_send()
            mk(rsL.at[s], rsL.at[s], rsL_s.at[s], rsL_r.at[s],
               left).wait_send()
            mk(agR.at[s], agR.at[s], agR_s.at[s], agR_r.at[s],
               right).wait_send()
            mk(agL.at[s], agL.at[s], agL_s.at[s], agL_r.at[s],
               left).wait_send()

    comm = pltpu.VMEM((n_steps, chunk, h2), jnp.float32)
    return pl.pallas_call(
        body,
        out_shape=jax.ShapeDtypeStruct((n_tok, h), jnp.float32),
        in_specs=[pl.BlockSpec(memory_space=pltpu.VMEM)] * 5,
        out_specs=pl.BlockSpec(memory_space=pltpu.VMEM),
        scratch_shapes=[
            pltpu.VMEM((n_tok, h), jnp.float32),
            comm, comm, comm, comm,
            pltpu.SemaphoreType.DMA((n_steps,)),
            pltpu.SemaphoreType.DMA((n_steps,)),
            pltpu.SemaphoreType.DMA((n_steps,)),
            pltpu.SemaphoreType.DMA((n_steps,)),
            pltpu.SemaphoreType.DMA((n_steps,)),
            pltpu.SemaphoreType.DMA((n_steps,)),
            pltpu.SemaphoreType.DMA((n_steps,)),
            pltpu.SemaphoreType.DMA((n_steps,)),
        ],
        compiler_params=pltpu.CompilerParams(collective_id=0),
    )(x, router_W, route_idx, expert_W, shared_W)


# device time: 71044 ns/iter; 1.7554x vs baseline; 1.2133x over previous
import jax
import jax.numpy as jnp
from jax import lax
from jax.experimental import pallas as pl
from jax.experimental.pallas import tpu as pltpu

N_DEV = 8


def kernel(x, router_W, route_idx, expert_W, shared_W):
    n_tok, d = x.shape
    n_exp = router_W.shape[1]
    e_loc, _, h = expert_W.shape
    chunk = n_tok // N_DEV
    h2 = h // 2

    def body(x_ref, rw_ref, idx_ref, ew_ref, sw_ref, out_ref,
             partial_ref, redA, redB, rA0, rA1, rA2, rB0, rB1, rB2,
             A_s, A_r, B_s, B_r, aA_s, aA_r, aB_s, aB_r):
        my = lax.axis_index("i")
        ell = my ^ ((my >> 1) & 1)
        b1 = ell & 1
        b2 = (ell >> 1) & 1
        b4 = (ell >> 2) & 1

        def logi(l):
            return l ^ ((l >> 1) & 1)

        nx = logi(ell ^ 1)
        ny = logi(ell ^ 2)
        nz = logi(ell ^ 4)

        def rows(c, n=1):
            return pl.ds(c * chunk, n * chunk)

        A = pl.ds(0, h2)
        B = pl.ds(h2, h2)

        def mk(src, dst, ssem, rsem, dev):
            return pltpu.make_async_remote_copy(
                src_ref=src, dst_ref=dst, send_sem=ssem, recv_sem=rsem,
                device_id=(dev,), device_id_type=pl.DeviceIdType.MESH)

        xv = x_ref[:, :]

        scores = jnp.dot(xv, rw_ref[:, :], preferred_element_type=jnp.float32)
        m = jnp.max(scores, axis=-1, keepdims=True)
        p = jnp.exp(scores - m)
        probs = p / jnp.sum(p, axis=-1, keepdims=True)
        ridx = idx_ref[:, 0:1]
        e_ids = lax.broadcasted_iota(jnp.int32, (n_tok, n_exp), 1)
        p_sel = jnp.sum(jnp.where(e_ids == ridx, probs, 0.0),
                        axis=1, keepdims=True)

        acc = jnp.zeros((n_tok, h), jnp.float32)
        for k in range(e_loc):
            e_g = my * e_loc + k
            coeff = jnp.where(ridx == e_g, p_sel, 0.0)
            y = jnp.dot(xv, ew_ref[k], preferred_element_type=jnp.float32)
            acc = acc + coeff * y
        partial_ref[:, :] = acc

        out_ref[:, :] = jnp.dot(xv, sw_ref[:, :],
                                preferred_element_type=jnp.float32)

        barrier_sem = pltpu.get_barrier_semaphore()
        for nbr in (nx, ny, nz):
            pl.semaphore_signal(barrier_sem, inc=1, device_id=(nbr,),
                                device_id_type=pl.DeviceIdType.MESH)
        pl.semaphore_wait(barrier_sem, 3)

        kA0 = b4 * 4
        sA0 = (1 - b4) * 4
        kA1 = kA0 + b2 * 2
        sA1 = kA0 + (1 - b2) * 2
        sA2 = kA1 + (1 - b1)
        kB0a = b2 * 2
        kB0b = 4 + b2 * 2
        sB0a = (1 - b2) * 2
        sB0b = 4 + (1 - b2) * 2
        kB1a = kB0a + b1
        kB1b = kB0b + b1
        sB1a = kB0a + (1 - b1)
        sB1b = kB0b + (1 - b1)
        sB2 = (1 - b4) * 4 + b2 * 2 + b1

        mk(partial_ref.at[rows(sA0, 4), A], rA0,
           A_s.at[0], A_r.at[0], nz).start()
        mk(partial_ref.at[rows(sB0a, 2), B], rB0.at[pl.ds(0, 2 * chunk)],
           B_s.at[0], B_r.at[0], ny).start()
        mk(partial_ref.at[rows(sB0b, 2), B], rB0.at[pl.ds(2 * chunk, 2 * chunk)],
           B_s.at[1], B_r.at[1], ny).start()

        mk(rA0, rA0, A_s.at[0], A_r.at[0], nz).wait_recv()
        partial_ref[rows(kA0, 4), A] = partial_ref[rows(kA0, 4), A] + rA0[:, :]
        mk(partial_ref.at[rows(sA1, 2), A], rA1,
           A_s.at[1], A_r.at[1], ny).start()
        mk(rB0.at[pl.ds(0, 2 * chunk)], rB0.at[pl.ds(0, 2 * chunk)],
           B_s.at[0], B_r.at[0], ny).wait_recv()
        mk(rB0.at[pl.ds(2 * chunk, 2 * chunk)], rB0.at[pl.ds(2 * chunk, 2 * chunk)],
           B_s.at[1], B_r.at[1], ny).wait_recv()
        partial_ref[rows(kB0a, 2), B] = (
            partial_ref[rows(kB0a, 2), B] + rB0[0:2 * chunk, :])
        partial_ref[rows(kB0b, 2), B] = (
            partial_ref[rows(kB0b, 2), B] + rB0[2 * chunk:4 * chunk, :])
        mk(partial_ref.at[rows(sB1a), B], rB1.at[pl.ds(0, chunk)],
           B_s.at[2], B_r.at[2], nx).start()
        mk(partial_ref.at[rows(sB1b), B], rB1.at[pl.ds(chunk, chunk)],
           B_s.at[3], B_r.at[3], nx).start()

        mk(rA1, rA1, A_s.at[1], A_r.at[1], ny).wait_recv()
        partial_ref[rows(kA1, 2), A] = partial_ref[rows(kA1, 2), A] + rA1[:, :]
        mk(partial_ref.at[rows(sA2), A], rA2,
           A_s.at[2], A_r.at[2], nx).start()
        mk(rB1.at[pl.ds(0, chunk)], rB1.at[pl.ds(0, chunk)],
           B_s.at[2], B_r.at[2], nx).wait_recv()
        mk(rB1.at[pl.ds(chunk, chunk)], rB1.at[pl.ds(chunk, chunk)],
           B_s.at[3], B_r.at[3], nx).wait_recv()
        partial_ref[rows(kB1a), B] = (
            partial_ref[rows(kB1a), B] + rB1[0:chunk, :])
        partial_ref[rows(kB1b), B] = (
            partial_ref[rows(kB1b), B] + rB1[chunk:2 * chunk, :])
        mk(partial_ref.at[rows(sB2), B], rB2,
           B_s.at[4], B_r.at[4], nz).start()

        mk(rA2, rA2, A_s.at[2], A_r.at[2], nx).wait_recv()
        mk(rB2, rB2, B_s.at[4], B_r.at[4], nz).wait_recv()
        redA[rows(ell), :] = partial_ref[rows(ell), A] + rA2[:, :]
        redB[rows(ell), :] = partial_ref[rows(ell), B] + rB2[:, :]

        pairA = ell & ~1
        quadA = ell & ~3
        mk(redA.at[rows(ell)], redA.at[rows(ell)],
           aA_s.at[0], aA_r.at[0], nx).start()
        mk(redB.at[rows(ell)], redB.at[rows(ell)],
           aB_s.at[0], aB_r.at[0], nz).start()
        out_ref[rows(ell), A] = out_ref[rows(ell), A] + redA[rows(ell), :]
        out_ref[rows(ell), B] = out_ref[rows(ell), B] + redB[rows(ell), :]

        mk(redA.at[rows(ell)], redA.at[rows(ell)],
           aA_s.at[0], aA_r.at[0], nx).wait_recv()
        mk(redB.at[rows(ell)], redB.at[rows(ell)],
           aB_s.at[0], aB_r.at[0], nz).wait_recv()
        mk(redA.at[rows(pairA, 2)], redA.at[rows(pairA, 2)],
           aA_s.at[1], aA_r.at[1], ny).start()
        mk(redB.at[rows(ell)], redB.at[rows(ell)],
           aB_s.at[1], aB_r.at[1], nx).start()
        mk(redB.at[rows(ell ^ 4)], redB.at[rows(ell ^ 4)],
           aB_s.at[2], aB_r.at[2], nx).start()
        cA = ell ^ 1
        out_ref[rows(cA), A] = out_ref[rows(cA), A] + redA[rows(cA), :]
        cB = ell ^ 4
        out_ref[rows(cB), B] = out_ref[rows(cB), B] + redB[rows(cB), :]

        mk(redA.at[rows(pairA, 2)], redA.at[rows(pairA, 2)],
           aA_s.at[1], aA_r.at[1], ny).wait_recv()
        mk(redB.at[rows(ell)], redB.at[rows(ell)],
           aB_s.at[1], aB_r.at[1], nx).wait_recv()
        mk(redB.at[rows(ell ^ 4)], redB.at[rows(ell ^ 4)],
           aB_s.at[2], aB_r.at[2], nx).wait_recv()
        mk(redA.at[rows(quadA, 4)], redA.at[rows(quadA, 4)],
           aA_s.at[2], aA_r.at[2], nz).start()
        mk(redB.at[rows(pairA, 2)], redB.at[rows(pairA, 2)],
           aB_s.at[3], aB_r.at[3], ny).start()
        mk(redB.at[rows(pairA ^ 4, 2)], redB.at[rows(pairA ^ 4, 2)],
           aB_s.at[4], aB_r.at[4], ny).start()
        cA = pairA ^ 2
        out_ref[rows(cA, 2), A] = out_ref[rows(cA, 2), A] + redA[rows(cA, 2), :]
        cB = ell ^ 1
        out_ref[rows(cB), B] = out_ref[rows(cB), B] + redB[rows(cB), :]
        cB = ell ^ 1 ^ 4
        out_ref[rows(cB), B] = out_ref[rows(cB), B] + redB[rows(cB), :]

        mk(redA.at[rows(quadA, 4)], redA.at[rows(quadA, 4)],
           aA_s.at[2], aA_r.at[2], nz).wait_recv()
        mk(redB.at[rows(pairA, 2)], redB.at[rows(pairA, 2)],
           aB_s.at[3], aB_r.at[3], ny).wait_recv()
        mk(redB.at[rows(pairA ^ 4, 2)], redB.at[rows(pairA ^ 4, 2)],
           aB_s.at[4], aB_r.at[4], ny).wait_recv()
        cA = quadA ^ 4
        out_ref[rows(cA, 4), A] = out_ref[rows(cA, 4), A] + redA[rows(cA, 4), :]
        cB = pairA ^ 2
        out_ref[rows(cB, 2), B] = out_ref[rows(cB, 2), B] + redB[rows(cB, 2), :]
        cB = pairA ^ 2 ^ 4
        out_ref[rows(cB, 2), B] = out_ref[rows(cB, 2), B] + redB[rows(cB, 2), :]

        mk(rA0, rA0, A_s.at[0], A_r.at[0], nz).wait_send()
        mk(rA1, rA1, A_s.at[1], A_r.at[1], ny).wait_send()
        mk(rA2, rA2, A_s.at[2], A_r.at[2], nx).wait_send()
        mk(rB0.at[pl.ds(0, 2 * chunk)], rB0.at[pl.ds(0, 2 * chunk)],
           B_s.at[0], B_r.at[0], ny).wait_send()
        mk(rB0.at[pl.ds(0, 2 * chunk)], rB0.at[pl.ds(0, 2 * chunk)],
           B_s.at[1], B_r.at[1], ny).wait_send()
        mk(rB2, rB2, B_s.at[2], B_r.at[2], nx).wait_send()
        mk(rB2, rB2, B_s.at[3], B_r.at[3], nx).wait_send()
        mk(rB2, rB2, B_s.at[4], B_r.at[4], nz).wait_send()
        mk(redA.at[rows(ell)], redA.at[rows(ell)],
           aA_s.at[0], aA_r.at[0], nx).wait_send()
        mk(redA.at[rows(pairA, 2)], redA.at[rows(pairA, 2)],
           aA_s.at[1], aA_r.at[1], ny).wait_send()
        mk(redA.at[rows(quadA, 4)], redA.at[rows(quadA, 4)],
           aA_s.at[2], aA_r.at[2], nz).wait_send()
        mk(redB.at[rows(ell)], redB.at[rows(ell)],
           aB_s.at[0], aB_r.at[0], nz).wait_send()
        mk(redB.at[rows(ell)], redB.at[rows(ell)],
           aB_s.at[1], aB_r.at[1], nx).wait_send()
        mk(redB.at[rows(ell)], redB.at[rows(ell)],
           aB_s.at[2], aB_r.at[2], nx).wait_send()
        mk(redB.at[rows(pairA, 2)], redB.at[rows(pairA, 2)],
           aB_s.at[3], aB_r.at[3], ny).wait_send()
        mk(redB.at[rows(pairA, 2)], redB.at[rows(pairA, 2)],
           aB_s.at[4], aB_r.at[4], ny).wait_send()

    return pl.pallas_call(
        body,
        out_shape=jax.ShapeDtypeStruct((n_tok, h), jnp.float32),
        in_specs=[pl.BlockSpec(memory_space=pltpu.VMEM)] * 5,
        out_specs=pl.BlockSpec(memory_space=pltpu.VMEM),
        scratch_shapes=[
            pltpu.VMEM((n_tok, h), jnp.float32),
            pltpu.VMEM((n_tok, h2), jnp.float32),
            pltpu.VMEM((n_tok, h2), jnp.float32),
            pltpu.VMEM((4 * chunk, h2), jnp.float32),
            pltpu.VMEM((2 * chunk, h2), jnp.float32),
            pltpu.VMEM((chunk, h2), jnp.float32),
            pltpu.VMEM((4 * chunk, h2), jnp.float32),
            pltpu.VMEM((2 * chunk, h2), jnp.float32),
            pltpu.VMEM((chunk, h2), jnp.float32),
            pltpu.SemaphoreType.DMA((3,)),
            pltpu.SemaphoreType.DMA((3,)),
            pltpu.SemaphoreType.DMA((5,)),
            pltpu.SemaphoreType.DMA((5,)),
            pltpu.SemaphoreType.DMA((3,)),
            pltpu.SemaphoreType.DMA((3,)),
            pltpu.SemaphoreType.DMA((5,)),
            pltpu.SemaphoreType.DMA((5,)),
        ],
        compiler_params=pltpu.CompilerParams(collective_id=0),
    )(x, router_W, route_idx, expert_W, shared_W)


# device time: 67208 ns/iter; 1.8556x vs baseline; 1.0571x over previous
import jax
import jax.numpy as jnp
from jax import lax
from jax.experimental import pallas as pl
from jax.experimental.pallas import tpu as pltpu

N_DEV = 8


def kernel(x, router_W, route_idx, expert_W, shared_W):
    n_tok, d = x.shape
    n_exp = router_W.shape[1]
    e_loc, _, h = expert_W.shape
    chunk = n_tok // N_DEV
    h2 = h // 2

    def body(x_ref, rw_ref, idx_ref, ew_ref, sw_ref, out_ref,
             partial_ref, redA, redB, rA0, rA1, rA2, rB0, rB1, rB2, psel_ref,
             A_s, A_r, B_s, B_r, aA_s, aA_r, aB_s, aB_r):
        my = lax.axis_index("i")
        ell = my ^ ((my >> 1) & 1)
        b1 = ell & 1
        b2 = (ell >> 1) & 1
        b4 = (ell >> 2) & 1

        def logi(l):
            return l ^ ((l >> 1) & 1)

        nx = logi(ell ^ 1)
        ny = logi(ell ^ 2)
        nz = logi(ell ^ 4)

        def rows(c, n=1):
            return pl.ds(c * chunk, n * chunk)

        A = pl.ds(0, h2)
        B = pl.ds(h2, h2)

        def mk(src, dst, ssem, rsem, dev):
            return pltpu.make_async_remote_copy(
                src_ref=src, dst_ref=dst, send_sem=ssem, recv_sem=rsem,
                device_id=(dev,), device_id_type=pl.DeviceIdType.MESH)

        xv = x_ref[:, :]

        scores = jnp.dot(xv, rw_ref[:, :], preferred_element_type=jnp.float32)
        m = jnp.max(scores, axis=-1, keepdims=True)
        p = jnp.exp(scores - m)
        probs = p / jnp.sum(p, axis=-1, keepdims=True)
        ridx = idx_ref[:, 0:1]
        e_ids = lax.broadcasted_iota(jnp.int32, (n_tok, n_exp), 1)
        p_sel = jnp.sum(jnp.where(e_ids == ridx, probs, 0.0),
                        axis=1, keepdims=True)

        psel_ref[:, :] = p_sel

        def comp(rs_chunks, n_chunks, col0):
            rws = pl.ds(rs_chunks * chunk, n_chunks * chunk)
            xb = x_ref[rws, :]
            rb = idx_ref[rws, 0:1]
            pb = psel_ref[rws, :]
            accu = jnp.zeros((n_chunks * chunk, h2), jnp.float32)
            for k in range(e_loc):
                w = ew_ref[k, :, col0:col0 + h2]
                ck = jnp.where(rb == my * e_loc + k, pb, 0.0)
                accu = accu + ck * jnp.dot(
                    xb, w, preferred_element_type=jnp.float32)
            partial_ref[rws, col0:col0 + h2] = accu

        kA0 = b4 * 4
        sA0 = (1 - b4) * 4
        kA1 = kA0 + b2 * 2
        sA1 = kA0 + (1 - b2) * 2
        sA2 = kA1 + (1 - b1)
        kB0a = b2 * 2
        kB0b = 4 + b2 * 2
        sB0a = (1 - b2) * 2
        sB0b = 4 + (1 - b2) * 2
        kB1a = kB0a + b1
        kB1b = kB0b + b1
        sB1a = kB0a + (1 - b1)
        sB1b = kB0b + (1 - b1)
        sB2 = (1 - b4) * 4 + b2 * 2 + b1

        comp(sA0, 4, 0)
        comp(sB0a, 2, h2)
        comp(sB0b, 2, h2)

        barrier_sem = pltpu.get_barrier_semaphore()
        for nbr in (nx, ny, nz):
            pl.semaphore_signal(barrier_sem, inc=1, device_id=(nbr,),
                                device_id_type=pl.DeviceIdType.MESH)
        pl.semaphore_wait(barrier_sem, 3)

        mk(partial_ref.at[rows(sA0, 4), A], rA0,
           A_s.at[0], A_r.at[0], nz).start()
        mk(partial_ref.at[rows(sB0a, 2), B], rB0.at[pl.ds(0, 2 * chunk)],
           B_s.at[0], B_r.at[0], ny).start()
        mk(partial_ref.at[rows(sB0b, 2), B], rB0.at[pl.ds(2 * chunk, 2 * chunk)],
           B_s.at[1], B_r.at[1], ny).start()

        comp(kA0, 4, 0)
        comp(kB0a, 2, h2)
        comp(kB0b, 2, h2)

        mk(rA0, rA0, A_s.at[0], A_r.at[0], nz).wait_recv()
        partial_ref[rows(kA0, 4), A] = partial_ref[rows(kA0, 4), A] + rA0[:, :]
        mk(partial_ref.at[rows(sA1, 2), A], rA1,
           A_s.at[1], A_r.at[1], ny).start()
        mk(rB0.at[pl.ds(0, 2 * chunk)], rB0.at[pl.ds(0, 2 * chunk)],
           B_s.at[0], B_r.at[0], ny).wait_recv()
        mk(rB0.at[pl.ds(2 * chunk, 2 * chunk)], rB0.at[pl.ds(2 * chunk, 2 * chunk)],
           B_s.at[1], B_r.at[1], ny).wait_recv()
        partial_ref[rows(kB0a, 2), B] = (
            partial_ref[rows(kB0a, 2), B] + rB0[0:2 * chunk, :])
        partial_ref[rows(kB0b, 2), B] = (
            partial_ref[rows(kB0b, 2), B] + rB0[2 * chunk:4 * chunk, :])
        mk(partial_ref.at[rows(sB1a), B], rB1.at[pl.ds(0, chunk)],
           B_s.at[2], B_r.at[2], nx).start()
        mk(partial_ref.at[rows(sB1b), B], rB1.at[pl.ds(chunk, chunk)],
           B_s.at[3], B_r.at[3], nx).start()

        out_ref[:, :] = jnp.dot(xv, sw_ref[:, :],
                                preferred_element_type=jnp.float32)

        mk(rA1, rA1, A_s.at[1], A_r.at[1], ny).wait_recv()
        partial_ref[rows(kA1, 2), A] = partial_ref[rows(kA1, 2), A] + rA1[:, :]
        mk(partial_ref.at[rows(sA2), A], rA2,
           A_s.at[2], A_r.at[2], nx).start()
        mk(rB1.at[pl.ds(0, chunk)], rB1.at[pl.ds(0, chunk)],
           B_s.at[2], B_r.at[2], nx).wait_recv()
        mk(rB1.at[pl.ds(chunk, chunk)], rB1.at[pl.ds(chunk, chunk)],
           B_s.at[3], B_r.at[3], nx).wait_recv()
        partial_ref[rows(kB1a), B] = (
            partial_ref[rows(kB1a), B] + rB1[0:chunk, :])
        partial_ref[rows(kB1b), B] = (
            partial_ref[rows(kB1b), B] + rB1[chunk:2 * chunk, :])
        mk(partial_ref.at[rows(sB2), B], rB2,
           B_s.at[4], B_r.at[4], nz).start()

        mk(rA2, rA2, A_s.at[2], A_r.at[2], nx).wait_recv()
        mk(rB2, rB2, B_s.at[4], B_r.at[4], nz).wait_recv()
        redA[rows(ell), :] = partial_ref[rows(ell), A] + rA2[:, :]
        redB[rows(ell), :] = partial_ref[rows(ell), B] + rB2[:, :]

        pairA = ell & ~1
        quadA = ell & ~3
        mk(redA.at[rows(ell)], redA.at[rows(ell)],
           aA_s.at[0], aA_r.at[0], nx).start()
        mk(redB.at[rows(ell)], redB.at[rows(ell)],
           aB_s.at[0], aB_r.at[0], nz).start()
        out_ref[rows(ell), A] = out_ref[rows(ell), A] + redA[rows(ell), :]
        out_ref[rows(ell), B] = out_ref[rows(ell), B] + redB[rows(ell), :]

        mk(redA.at[rows(ell)], redA.at[rows(ell)],
           aA_s.at[0], aA_r.at[0], nx).wait_recv()
        mk(redB.at[rows(ell)], redB.at[rows(ell)],
           aB_s.at[0], aB_r.at[0], nz).wait_recv()
        mk(redA.at[rows(pairA, 2)], redA.at[rows(pairA, 2)],
           aA_s.at[1], aA_r.at[1], ny).start()
        mk(redB.at[rows(ell)], redB.at[rows(ell)],
           aB_s.at[1], aB_r.at[1], nx).start()
        mk(redB.at[rows(ell ^ 4)], redB.at[rows(ell ^ 4)],
           aB_s.at[2], aB_r.at[2], nx).start()
        cA = ell ^ 1
        out_ref[rows(cA), A] = out_ref[rows(cA), A] + redA[rows(cA), :]
        cB = ell ^ 4
        out_ref[rows(cB), B] = out_ref[rows(cB), B] + redB[rows(cB), :]

        mk(redA.at[rows(pairA, 2)], redA.at[rows(pairA, 2)],
           aA_s.at[1], aA_r.at[1], ny).wait_recv()
        mk(redB.at[rows(ell)], redB.at[rows(ell)],
           aB_s.at[1], aB_r.at[1], nx).wait_recv()
        mk(redB.at[rows(ell ^ 4)], redB.at[rows(ell ^ 4)],
           aB_s.at[2], aB_r.at[2], nx).wait_recv()
        mk(redA.at[rows(quadA, 4)], redA.at[rows(quadA, 4)],
           aA_s.at[2], aA_r.at[2], nz).start()
        mk(redB.at[rows(pairA, 2)], redB.at[rows(pairA, 2)],
           aB_s.at[3], aB_r.at[3], ny).start()
        mk(redB.at[rows(pairA ^ 4, 2)], redB.at[rows(pairA ^ 4, 2)],
           aB_s.at[4], aB_r.at[4], ny).start()
        cA = pairA ^ 2
        out_ref[rows(cA, 2), A] = out_ref[rows(cA, 2), A] + redA[rows(cA, 2), :]
        cB = ell ^ 1
        out_ref[rows(cB), B] = out_ref[rows(cB), B] + redB[rows(cB), :]
        cB = ell ^ 1 ^ 4
        out_ref[rows(cB), B] = out_ref[rows(cB), B] + redB[rows(cB), :]

        mk(redA.at[rows(quadA, 4)], redA.at[rows(quadA, 4)],
           aA_s.at[2], aA_r.at[2], nz).wait_recv()
        mk(redB.at[rows(pairA, 2)], redB.at[rows(pairA, 2)],
           aB_s.at[3], aB_r.at[3], ny).wait_recv()
        mk(redB.at[rows(pairA ^ 4, 2)], redB.at[rows(pairA ^ 4, 2)],
           aB_s.at[4], aB_r.at[4], ny).wait_recv()
        cA = quadA ^ 4
        out_ref[rows(cA, 4), A] = out_ref[rows(cA, 4), A] + redA[rows(cA, 4), :]
        cB = pairA ^ 2
        out_ref[rows(cB, 2), B] = out_ref[rows(cB, 2), B] + redB[rows(cB, 2), :]
        cB = pairA ^ 2 ^ 4
        out_ref[rows(cB, 2), B] = out_ref[rows(cB, 2), B] + redB[rows(cB, 2), :]

        mk(rA0, rA0, A_s.at[0], A_r.at[0], nz).wait_send()
        mk(rA1, rA1, A_s.at[1], A_r.at[1], ny).wait_send()
        mk(rA2, rA2, A_s.at[2], A_r.at[2], nx).wait_send()
        mk(rB0.at[pl.ds(0, 2 * chunk)], rB0.at[pl.ds(0, 2 * chunk)],
           B_s.at[0], B_r.at[0], ny).wait_send()
        mk(rB0.at[pl.ds(0, 2 * chunk)], rB0.at[pl.ds(0, 2 * chunk)],
           B_s.at[1], B_r.at[1], ny).wait_send()
        mk(rB2, rB2, B_s.at[2], B_r.at[2], nx).wait_send()
        mk(rB2, rB2, B_s.at[3], B_r.at[3], nx).wait_send()
        mk(rB2, rB2, B_s.at[4], B_r.at[4], nz).wait_send()
        mk(redA.at[rows(ell)], redA.at[rows(ell)],
           aA_s.at[0], aA_r.at[0], nx).wait_send()
        mk(redA.at[rows(pairA, 2)], redA.at[rows(pairA, 2)],
           aA_s.at[1], aA_r.at[1], ny).wait_send()
        mk(redA.at[rows(quadA, 4)], redA.at[rows(quadA, 4)],
           aA_s.at[2], aA_r.at[2], nz).wait_send()
        mk(redB.at[rows(ell)], redB.at[rows(ell)],
           aB_s.at[0], aB_r.at[0], nz).wait_send()
        mk(redB.at[rows(ell)], redB.at[rows(ell)],
           aB_s.at[1], aB_r.at[1], nx).wait_send()
        mk(redB.at[rows(ell)], redB.at[rows(ell)],
           aB_s.at[2], aB_r.at[2], nx).wait_send()
        mk(redB.at[rows(pairA, 2)], redB.at[rows(pairA, 2)],
           aB_s.at[3], aB_r.at[3], ny).wait_send()
        mk(redB.at[rows(pairA, 2)], redB.at[rows(pairA, 2)],
           aB_s.at[4], aB_r.at[4], ny).wait_send()

    return pl.pallas_call(
        body,
        out_shape=jax.ShapeDtypeStruct((n_tok, h), jnp.float32),
        in_specs=[pl.BlockSpec(memory_space=pltpu.VMEM)] * 5,
        out_specs=pl.BlockSpec(memory_space=pltpu.VMEM),
        scratch_shapes=[
            pltpu.VMEM((n_tok, h), jnp.float32),
            pltpu.VMEM((n_tok, h2), jnp.float32),
            pltpu.VMEM((n_tok, h2), jnp.float32),
            pltpu.VMEM((4 * chunk, h2), jnp.float32),
            pltpu.VMEM((2 * chunk, h2), jnp.float32),
            pltpu.VMEM((chunk, h2), jnp.float32),
            pltpu.VMEM((4 * chunk, h2), jnp.float32),
            pltpu.VMEM((2 * chunk, h2), jnp.float32),
            pltpu.VMEM((chunk, h2), jnp.float32),
            pltpu.VMEM((n_tok, 1), jnp.float32),
            pltpu.SemaphoreType.DMA((3,)),
            pltpu.SemaphoreType.DMA((3,)),
            pltpu.SemaphoreType.DMA((5,)),
            pltpu.SemaphoreType.DMA((5,)),
            pltpu.SemaphoreType.DMA((3,)),
            pltpu.SemaphoreType.DMA((3,)),
            pltpu.SemaphoreType.DMA((5,)),
            pltpu.SemaphoreType.DMA((5,)),
        ],
        compiler_params=pltpu.CompilerParams(collective_id=0),
    )(x, router_W, route_idx, expert_W, shared_W)
